# Initial kernel scaffold; baseline (speedup 1.0000x reference)
import jax
import jax.numpy as jnp
from jax.experimental import pallas as pl


def kernel(x, edge_index, W1, a_src1, a_dst1, b1, W2, a_src2, a_dst2, b2, W3, a_src3, a_dst3, b3):
    raise NotImplementedError("write your pallas kernel here")


# speedup vs baseline: 24.3974x; 24.3974x over previous
import functools

import jax
import jax.numpy as jnp
from jax import lax
from jax.experimental import pallas as pl
from jax.experimental.pallas import tpu as pltpu
from jax.experimental.pallas import tpu_sc as plsc

N = 10000
E = 320000
D = 128

NC = 2
NS = 16
NW = NC * NS
EPW = E // NW
C = 80
NCHUNK = EPW // C
NP = 10240
RPT = NP // NS
LRELU_SLOPE = 0.2
EPS = 1e-16


def _proj_body(h_ref, w_ref, a_ref, xp_ref, asrc_ref, adst_ref, m_ref):
    xp = jnp.dot(h_ref[...], w_ref[...], preferred_element_type=jnp.float32)
    xp_ref[...] = xp
    al = jnp.dot(xp, a_ref[...], preferred_element_type=jnp.float32)
    asrc = al[:, 0]
    adst = al[:, 1]
    asrc_ref[...] = asrc
    adst_ref[...] = adst
    m = jnp.max(asrc) + jnp.max(adst)
    m_ref[...] = jnp.full((16,), m, jnp.float32)


def _proj(h, w, a2):
    return pl.pallas_call(
        _proj_body,
        out_shape=[
            jax.ShapeDtypeStruct((N, D), jnp.float32),
            jax.ShapeDtypeStruct((N,), jnp.float32),
            jax.ShapeDtypeStruct((N,), jnp.float32),
            jax.ShapeDtypeStruct((16,), jnp.float32),
        ],
    )(h, w, a2)


def _self_loop_combine(acc_ref, den_ref, xp_ref, asrc_ref, adst_ref, m_ref,
                       b_ref):
    num = acc_ref[0, :N, :] + acc_ref[1, :N, :]
    den = jnp.sum(den_ref[...], axis=0)[:, None]
    es = asrc_ref[...] + adst_ref[...]
    es = jnp.where(es >= 0.0, es, LRELU_SLOPE * es)
    exs = jnp.exp(es - m_ref[0])
    num = num + exs[:, None] * xp_ref[...]
    den = den + exs[:, None]
    return jnp.maximum(num / (den + EPS) + b_ref[...], 0.0)


def _combine_mid_body(acc_ref, den_ref, xp_ref, asrc_ref, adst_ref, m_ref,
                      b_ref, w_ref, a_ref, xp2_ref, asrc2_ref, adst2_ref,
                      m2_ref):
    h = _self_loop_combine(acc_ref, den_ref, xp_ref, asrc_ref, adst_ref,
                           m_ref, b_ref)
    _proj_body(_Val(h), w_ref, a_ref, xp2_ref, asrc2_ref, adst2_ref, m2_ref)


class _Val:
    def __init__(self, v):
        self._v = v

    def __getitem__(self, idx):
        return self._v[idx] if idx is not Ellipsis else self._v


def _combine_mid(acc, den, xp, asrc, adst, m16, b, w_next, a_next):
    specs = [
        pl.BlockSpec(memory_space=pltpu.MemorySpace.VMEM),
        pl.BlockSpec(memory_space=pltpu.MemorySpace.VMEM),
        pl.BlockSpec(memory_space=pltpu.MemorySpace.VMEM),
        pl.BlockSpec(memory_space=pltpu.MemorySpace.VMEM),
        pl.BlockSpec(memory_space=pltpu.MemorySpace.VMEM),
        pl.BlockSpec(memory_space=pltpu.MemorySpace.SMEM),
        pl.BlockSpec(memory_space=pltpu.MemorySpace.VMEM),
        pl.BlockSpec(memory_space=pltpu.MemorySpace.VMEM),
        pl.BlockSpec(memory_space=pltpu.MemorySpace.VMEM),
    ]
    return pl.pallas_call(
        _combine_mid_body,
        in_specs=specs,
        out_shape=[
            jax.ShapeDtypeStruct((N, D), jnp.float32),
            jax.ShapeDtypeStruct((N,), jnp.float32),
            jax.ShapeDtypeStruct((N,), jnp.float32),
            jax.ShapeDtypeStruct((16,), jnp.float32),
        ],
    )(acc, den, xp, asrc, adst, m16, b, w_next, a_next)


def _combine_last_body(acc_ref, den_ref, xp_ref, asrc_ref, adst_ref, m_ref,
                       b_ref, out_ref):
    out_ref[...] = _self_loop_combine(acc_ref, den_ref, xp_ref, asrc_ref,
                                      adst_ref, m_ref, b_ref)


def _combine_last(acc, den, xp, asrc, adst, m16, b):
    specs = [
        pl.BlockSpec(memory_space=pltpu.MemorySpace.VMEM),
        pl.BlockSpec(memory_space=pltpu.MemorySpace.VMEM),
        pl.BlockSpec(memory_space=pltpu.MemorySpace.VMEM),
        pl.BlockSpec(memory_space=pltpu.MemorySpace.VMEM),
        pl.BlockSpec(memory_space=pltpu.MemorySpace.VMEM),
        pl.BlockSpec(memory_space=pltpu.MemorySpace.SMEM),
        pl.BlockSpec(memory_space=pltpu.MemorySpace.VMEM),
    ]
    return pl.pallas_call(
        _combine_last_body,
        in_specs=specs,
        out_shape=jax.ShapeDtypeStruct((N, D), jnp.float32),
    )(acc, den, xp, asrc, adst, m16, b)


def _sc_edge_body(xp_hbm, asrc_hbm, adst_hbm, m_hbm, src_hbm, dst_hbm,
                  acc_hbm, den_hbm, asrc_v, adst_v, m_v, den_v, src_i,
                  dst_i, ex_v, rows_v, acc_sh, sem):
    cid = lax.axis_index("c")
    sid = lax.axis_index("s")

    pltpu.sync_copy(asrc_hbm, asrc_v)
    pltpu.sync_copy(adst_hbm, adst_v)
    pltpu.sync_copy(m_hbm, m_v)

    zeros16 = jnp.zeros((16,), jnp.float32)

    def zero_den(i, carry):
        den_v[pl.ds(i * 16, 16)] = zeros16
        return carry

    lax.fori_loop(0, N // 16, zero_den, 0)

    def zero_row(r, carry):
        for j in range(D // 16):
            rows_v[r, pl.ds(j * 16, 16)] = zeros16
        return carry

    lax.fori_loop(0, C, zero_row, 0)
    zbase = sid * RPT
    for t in range(RPT // C):
        pltpu.sync_copy(rows_v, acc_sh.at[pl.ds(zbase + t * C, C)])
    plsc.subcore_barrier()

    wbase = (sid * NC + cid) * EPW
    mv = m_v[...]

    def chunk(k, carry):
        base = pl.multiple_of(wbase + k * C, 8)
        pltpu.sync_copy(src_hbm.at[pl.ds(base, C)], src_i)
        pltpu.sync_copy(dst_hbm.at[pl.ds(base, C)], dst_i)
        pltpu.async_copy(xp_hbm.at[src_i], rows_v, sem).wait()

        for g in range(C // 16):
            sv = src_i[pl.ds(g * 16, 16)]
            dv = dst_i[pl.ds(g * 16, 16)]
            e = plsc.load_gather(asrc_v, [sv]) + plsc.load_gather(adst_v, [dv])
            e = jnp.where(e >= 0.0, e, LRELU_SLOPE * e)
            ex = jnp.exp(e - mv)
            ex_v[pl.ds(g * 16, 16)] = ex
            plsc.addupdate_scatter(den_v, [dv], ex)

        def scale_row(r, carry):
            s = plsc.load_gather(ex_v, [jnp.full((16,), r, jnp.int32)])
            for j in range(D // 16):
                rows_v[r, pl.ds(j * 16, 16)] = rows_v[r, pl.ds(j * 16, 16)] * s
            return carry

        lax.fori_loop(0, C, scale_row, 0)

        pltpu.sync_copy(rows_v, acc_sh.at[dst_i], add=True)
        return carry

    lax.fori_loop(0, NCHUNK, chunk, 0)
    plsc.subcore_barrier()

    pltpu.sync_copy(acc_sh.at[pl.ds(zbase, RPT)],
                    acc_hbm.at[cid, pl.ds(zbase, RPT)])
    pltpu.sync_copy(den_v, den_hbm.at[cid, sid])


@functools.cache
def _make_sc_edge():
    return functools.partial(
        pl.kernel,
        out_type=[
            jax.ShapeDtypeStruct((NC, NP, D), jnp.float32),
            jax.ShapeDtypeStruct((NC, NS, N), jnp.float32),
        ],
        mesh=plsc.VectorSubcoreMesh(core_axis_name="c", subcore_axis_name="s",
                                    num_cores=NC, num_subcores=NS),
        compiler_params=pltpu.CompilerParams(needs_layout_passes=False),
        scratch_types=[
            pltpu.VMEM((N,), jnp.float32),
            pltpu.VMEM((N,), jnp.float32),
            pltpu.VMEM((16,), jnp.float32),
            pltpu.VMEM((N,), jnp.float32),
            pltpu.VMEM((C,), jnp.int32),
            pltpu.VMEM((C,), jnp.int32),
            pltpu.VMEM((C,), jnp.float32),
            pltpu.VMEM((C, D), jnp.float32),
            pltpu.VMEM_SHARED((NP, D), jnp.float32),
            pltpu.SemaphoreType.DMA,
        ],
    )(_sc_edge_body)


def _sc_edge(xp, asrc, adst, m16, src, dst):
    return _make_sc_edge()(xp, asrc, adst, m16, src, dst)


def kernel(x, edge_index, W1, a_src1, a_dst1, b1, W2, a_src2, a_dst2, b2,
           W3, a_src3, a_dst3, b3):
    src = edge_index[0].astype(jnp.int32)
    dst = edge_index[1].astype(jnp.int32)
    a1 = jnp.stack([a_src1, a_dst1], axis=1)
    a2 = jnp.stack([a_src2, a_dst2], axis=1)
    a3 = jnp.stack([a_src3, a_dst3], axis=1)
    b1r = b1.reshape(1, D)
    b2r = b2.reshape(1, D)
    b3r = b3.reshape(1, D)

    xp, asrc, adst, m16 = _proj(x, W1, a1)
    acc, den = _sc_edge(xp, asrc, adst, m16, src, dst)
    den = den.reshape(NC * NS, N)
    xp, asrc, adst, m16 = _combine_mid(acc, den, xp, asrc, adst, m16, b1r,
                                       W2, a2)
    acc, den = _sc_edge(xp, asrc, adst, m16, src, dst)
    den = den.reshape(NC * NS, N)
    xp, asrc, adst, m16 = _combine_mid(acc, den, xp, asrc, adst, m16, b2r,
                                       W3, a3)
    acc, den = _sc_edge(xp, asrc, adst, m16, src, dst)
    den = den.reshape(NC * NS, N)
    return _combine_last(acc, den, xp, asrc, adst, m16, b3r)


# speedup vs baseline: 48.4407x; 1.9855x over previous
import functools

import jax
import jax.numpy as jnp
from jax import lax
from jax.experimental import pallas as pl
from jax.experimental.pallas import tpu as pltpu
from jax.experimental.pallas import tpu_sc as plsc

N = 10000
E = 320000
D = 128

NC = 2
NS = 16
NW = NC * NS
EPW = E // NW
C = 16
NCHUNK = EPW // C
NP = 10112
RPT = NP // NS
LRELU_SLOPE = 0.2
EPS = 1e-16


def _proj_body(h_ref, w_ref, a_ref, xp_ref, asrc_ref, adst_ref, m_ref):
    xp = jnp.dot(h_ref[...], w_ref[...], preferred_element_type=jnp.float32)
    xp_ref[...] = xp
    al = jnp.dot(xp, a_ref[...], preferred_element_type=jnp.float32)
    asrc = al[:, 0]
    adst = al[:, 1]
    asrc_ref[...] = asrc
    adst_ref[...] = adst
    m = jnp.max(asrc) + jnp.max(adst)
    m_ref[...] = jnp.full((16,), m, jnp.float32)


def _proj(h, w, a2):
    return pl.pallas_call(
        _proj_body,
        out_shape=[
            jax.ShapeDtypeStruct((N, D), jnp.float32),
            jax.ShapeDtypeStruct((N,), jnp.float32),
            jax.ShapeDtypeStruct((N,), jnp.float32),
            jax.ShapeDtypeStruct((16,), jnp.float32),
        ],
    )(h, w, a2)


def _self_loop_combine(acc_ref, den_ref, xp_ref, asrc_ref, adst_ref, m_ref,
                       b_ref):
    num = acc_ref[0, :N, :] + acc_ref[1, :N, :]
    den = jnp.sum(den_ref[...], axis=0)[:, None]
    es = asrc_ref[...] + adst_ref[...]
    es = jnp.where(es >= 0.0, es, LRELU_SLOPE * es)
    exs = jnp.exp(es - m_ref[0])
    num = num + exs[:, None] * xp_ref[...]
    den = den + exs[:, None]
    return jnp.maximum(num / (den + EPS) + b_ref[...], 0.0)


def _combine_mid_body(acc_ref, den_ref, xp_ref, asrc_ref, adst_ref, m_ref,
                      b_ref, w_ref, a_ref, xp2_ref, asrc2_ref, adst2_ref,
                      m2_ref):
    h = _self_loop_combine(acc_ref, den_ref, xp_ref, asrc_ref, adst_ref,
                           m_ref, b_ref)
    _proj_body(_Val(h), w_ref, a_ref, xp2_ref, asrc2_ref, adst2_ref, m2_ref)


class _Val:
    def __init__(self, v):
        self._v = v

    def __getitem__(self, idx):
        return self._v[idx] if idx is not Ellipsis else self._v


def _combine_mid(acc, den, xp, asrc, adst, m16, b, w_next, a_next):
    specs = [
        pl.BlockSpec(memory_space=pltpu.MemorySpace.VMEM),
        pl.BlockSpec(memory_space=pltpu.MemorySpace.VMEM),
        pl.BlockSpec(memory_space=pltpu.MemorySpace.VMEM),
        pl.BlockSpec(memory_space=pltpu.MemorySpace.VMEM),
        pl.BlockSpec(memory_space=pltpu.MemorySpace.VMEM),
        pl.BlockSpec(memory_space=pltpu.MemorySpace.SMEM),
        pl.BlockSpec(memory_space=pltpu.MemorySpace.VMEM),
        pl.BlockSpec(memory_space=pltpu.MemorySpace.VMEM),
        pl.BlockSpec(memory_space=pltpu.MemorySpace.VMEM),
    ]
    return pl.pallas_call(
        _combine_mid_body,
        in_specs=specs,
        out_shape=[
            jax.ShapeDtypeStruct((N, D), jnp.float32),
            jax.ShapeDtypeStruct((N,), jnp.float32),
            jax.ShapeDtypeStruct((N,), jnp.float32),
            jax.ShapeDtypeStruct((16,), jnp.float32),
        ],
    )(acc, den, xp, asrc, adst, m16, b, w_next, a_next)


def _combine_last_body(acc_ref, den_ref, xp_ref, asrc_ref, adst_ref, m_ref,
                       b_ref, out_ref):
    out_ref[...] = _self_loop_combine(acc_ref, den_ref, xp_ref, asrc_ref,
                                      adst_ref, m_ref, b_ref)


def _combine_last(acc, den, xp, asrc, adst, m16, b):
    specs = [
        pl.BlockSpec(memory_space=pltpu.MemorySpace.VMEM),
        pl.BlockSpec(memory_space=pltpu.MemorySpace.VMEM),
        pl.BlockSpec(memory_space=pltpu.MemorySpace.VMEM),
        pl.BlockSpec(memory_space=pltpu.MemorySpace.VMEM),
        pl.BlockSpec(memory_space=pltpu.MemorySpace.VMEM),
        pl.BlockSpec(memory_space=pltpu.MemorySpace.SMEM),
        pl.BlockSpec(memory_space=pltpu.MemorySpace.VMEM),
    ]
    return pl.pallas_call(
        _combine_last_body,
        in_specs=specs,
        out_shape=jax.ShapeDtypeStruct((N, D), jnp.float32),
    )(acc, den, xp, asrc, adst, m16, b)


R = 8


def _sc_edge_body(xp_hbm, asrc_hbm, adst_hbm, m_hbm, src_hbm, dst_hbm,
                  acc_hbm, den_hbm, asrc_v, adst_v, m_v, den_v, rows_v,
                  *rest):
    cid = lax.axis_index("c")
    sid = lax.axis_index("s")
    srci = rest[0:R]
    dsti = rest[R:2 * R]
    exi = rest[2 * R:3 * R]
    acc_sh = rest[3 * R]
    isem = rest[3 * R + 1:4 * R + 1]
    gsem = rest[4 * R + 1:5 * R + 1]
    ssem = rest[5 * R + 1:6 * R + 1]
    zeros16 = jnp.zeros((16,), jnp.float32)

    wbase = (sid * NC + cid) * EPW
    pltpu.sync_copy(asrc_hbm, asrc_v)
    pltpu.sync_copy(adst_hbm, adst_v)
    pltpu.sync_copy(m_hbm, m_v)

    def zero_den(i, carry):
        den_v[pl.ds(i * 16, 16)] = zeros16
        return carry

    lax.fori_loop(0, N // 16, zero_den, 0)

    def zero_row(r, carry):
        for j in range(D // 16):
            rows_v[r, pl.ds(j * 16, 16)] = zeros16
        return carry

    lax.fori_loop(0, R * C, zero_row, 0)
    zbase = sid * RPT
    for t in range(RPT // (R * C)):
        pltpu.sync_copy(rows_v, acc_sh.at[pl.ds(zbase + t * R * C, R * C)])
    zrem = RPT - (RPT // (R * C)) * (R * C)
    if zrem:
        pltpu.sync_copy(rows_v.at[pl.ds(0, zrem)],
                        acc_sh.at[pl.ds(zbase + RPT - zrem, zrem)])
    plsc.subcore_barrier()

    mv = m_v[...]

    def rows_at(i):
        return rows_v.at[pl.ds(i * C, C)]

    def drain_rows(sem, i):
        pltpu.make_async_copy(xp_hbm.at[pl.ds(0, C)], rows_at(i), sem).wait()

    def drain_idx(sem, i):
        pltpu.make_async_copy(src_hbm.at[pl.ds(0, C)], srci[i], sem).wait()
        pltpu.make_async_copy(dst_hbm.at[pl.ds(0, C)], dsti[i], sem).wait()

    def prefetch_idx(i, k):
        base = pl.multiple_of(wbase + k * C, 8)
        pltpu.async_copy(src_hbm.at[pl.ds(base, C)], srci[i], isem[i])
        pltpu.async_copy(dst_hbm.at[pl.ds(base, C)], dsti[i], isem[i])

    def process(i, k):
        drain_rows(gsem[i], i)
        sv = srci[i][...]
        dv = dsti[i][...]
        e = plsc.load_gather(asrc_v, [sv]) + plsc.load_gather(adst_v, [dv])
        e = jnp.where(e >= 0.0, e, LRELU_SLOPE * e)
        ex = jnp.exp(e - mv)
        plsc.addupdate_scatter(den_v, [dv], ex)

        @plsc.parallel_loop(0, C, 1, unroll=8)
        def scale_row(r):
            s = ex.at[jnp.full((16,), r, jnp.int32)].get(
                mode="promise_in_bounds")
            rb = i * C + r
            for j in range(D // 16):
                rows_v[rb, pl.ds(j * 16, 16)] = (
                    rows_v[rb, pl.ds(j * 16, 16)] * s)
        pltpu.async_copy(rows_at(i), acc_sh.at[dsti[i]], ssem[i], add=True)

    def make_branch(a):
        b = (a + 2) % R
        c = (a + 4) % R

        def branch(k):
            @pl.when(k >= 4)
            def _():
                drain_rows(ssem[c], c)

            @pl.when(k + 4 < NCHUNK)
            def _():
                prefetch_idx(c, k + 4)

            @pl.when(k + 2 < NCHUNK)
            def _():
                drain_idx(isem[b], b)
                pltpu.async_copy(xp_hbm.at[srci[b]], rows_at(b), gsem[b])

            process(a, k)

        return branch

    branches = [make_branch(a) for a in range(R)]

    for i in range(4):
        prefetch_idx(i, i)
    for i in range(2):
        drain_idx(isem[i], i)
        pltpu.async_copy(xp_hbm.at[srci[i]], rows_at(i), gsem[i])

    def chunk(k, carry):
        lax.switch(k % R, branches, k)
        return carry

    lax.fori_loop(0, NCHUNK, chunk, 0)
    for m in range(NCHUNK - 4, NCHUNK):
        drain_rows(ssem[m % R], m % R)
    plsc.subcore_barrier()

    pltpu.sync_copy(acc_sh.at[pl.ds(zbase, RPT)],
                    acc_hbm.at[cid, pl.ds(zbase, RPT)])
    pltpu.sync_copy(den_v, den_hbm.at[cid, sid])


@functools.cache
def _make_sc_edge():
    return functools.partial(
        pl.kernel,
        out_type=[
            jax.ShapeDtypeStruct((NC, NP, D), jnp.float32),
            jax.ShapeDtypeStruct((NC, NS, N), jnp.float32),
        ],
        mesh=plsc.VectorSubcoreMesh(core_axis_name="c", subcore_axis_name="s",
                                    num_cores=NC, num_subcores=NS),
        compiler_params=pltpu.CompilerParams(needs_layout_passes=False),
        scratch_types=(
            [
                pltpu.VMEM((N,), jnp.float32),
                pltpu.VMEM((N,), jnp.float32),
                pltpu.VMEM((16,), jnp.float32),
                pltpu.VMEM((N,), jnp.float32),
                pltpu.VMEM((R * C, D), jnp.float32),
            ]
            + [pltpu.VMEM((C,), jnp.int32) for _ in range(2 * R)]
            + [pltpu.VMEM((C,), jnp.float32) for _ in range(R)]
            + [pltpu.VMEM_SHARED((NP, D), jnp.float32)]
            + [pltpu.SemaphoreType.DMA for _ in range(3 * R)]
        ),
    )(_sc_edge_body)


def _sc_edge(xp, asrc, adst, m16, src, dst):
    return _make_sc_edge()(xp, asrc, adst, m16, src, dst)


def kernel(x, edge_index, W1, a_src1, a_dst1, b1, W2, a_src2, a_dst2, b2,
           W3, a_src3, a_dst3, b3):
    src = edge_index[0].astype(jnp.int32)
    dst = edge_index[1].astype(jnp.int32)
    a1 = jnp.stack([a_src1, a_dst1], axis=1)
    a2 = jnp.stack([a_src2, a_dst2], axis=1)
    a3 = jnp.stack([a_src3, a_dst3], axis=1)
    b1r = b1.reshape(1, D)
    b2r = b2.reshape(1, D)
    b3r = b3.reshape(1, D)

    xp, asrc, adst, m16 = _proj(x, W1, a1)
    acc, den = _sc_edge(xp, asrc, adst, m16, src, dst)
    den = den.reshape(NC * NS, N)
    xp, asrc, adst, m16 = _combine_mid(acc, den, xp, asrc, adst, m16, b1r,
                                       W2, a2)
    acc, den = _sc_edge(xp, asrc, adst, m16, src, dst)
    den = den.reshape(NC * NS, N)
    xp, asrc, adst, m16 = _combine_mid(acc, den, xp, asrc, adst, m16, b2r,
                                       W3, a3)
    acc, den = _sc_edge(xp, asrc, adst, m16, src, dst)
    den = den.reshape(NC * NS, N)
    return _combine_last(acc, den, xp, asrc, adst, m16, b3r)
